# Initial kernel scaffold; baseline (speedup 1.0000x reference)
import jax
import jax.numpy as jnp
from jax.experimental import pallas as pl


def kernel(x, stem_w, stem_g, stem_b, ds_w, b0_w_sc, b0_w1, b0_w2, b0_w3, b0_g1, b0_b1, b0_g2, b0_b2, b0_g3, b0_b3, b1_w_sc, b1_w1, b1_w2, b1_w3, b1_g1, b1_b1, b1_g2, b1_b2, b1_g3, b1_b3, b2_w_sc, b2_w1, b2_w2, b2_w3, b2_g1, b2_b1, b2_g2, b2_b2, b2_g3, b2_b3):
    raise NotImplementedError("write your pallas kernel here")


# speedup vs baseline: 1.1222x; 1.1222x over previous
import jax
import jax.numpy as jnp
from jax import lax
from jax.experimental import pallas as pl
from jax.experimental.pallas import tpu as pltpu

_EPS = 1e-5


def _ceil_to(v, m):
    return ((v + m - 1) // m) * m


def _bn_scale_shift(y, g, b):
    mean = jnp.mean(y, axis=0, keepdims=True)
    var = jnp.mean(y * y, axis=0, keepdims=True) - mean * mean
    scale = g * lax.rsqrt(var + _EPS)
    shift = b - mean * scale
    return scale, shift


def _stem_body(p_ref, w_ref, gb_ref, o_ref):
    y = jnp.dot(p_ref[...], w_ref[...], preferred_element_type=jnp.float32)
    scale, shift = _bn_scale_shift(y, gb_ref[0:1], gb_ref[1:2])
    o_ref[...] = jnp.maximum(y * scale + shift, 0.0).astype(jnp.bfloat16)


def _stem_call(p1, w_s, gb_s):
    m1, kp1 = p1.shape
    cs = w_s.shape[1]
    ch = cs // 2
    return pl.pallas_call(
        _stem_body,
        grid=(2,),
        in_specs=[
            pl.BlockSpec((m1, kp1), lambda j: (0, 0)),
            pl.BlockSpec((kp1, ch), lambda j: (0, j)),
            pl.BlockSpec((8, ch), lambda j: (0, j)),
        ],
        out_specs=pl.BlockSpec((m1, ch), lambda j: (0, j)),
        out_shape=jax.ShapeDtypeStruct((m1, cs), jnp.bfloat16),
        compiler_params=pltpu.CompilerParams(
            dimension_semantics=("parallel",)),
        cost_estimate=pl.CostEstimate(
            flops=int(2 * m1 * kp1 * cs), transcendentals=0,
            bytes_accessed=int(2 * (2 * m1 * kp1 + kp1 * cs + m1 * cs))),
    )(p1, w_s, gb_s)


def _make_main_body(n, h, c, nb):
    m = n * h * h

    def _body(aq_ref, wds_ref, wcat_ref, w2_ref, w3_ref, gb_ref, o_ref,
              qp_ref, hs_ref):
        qp_ref[...] = jnp.zeros(qp_ref.shape, jnp.bfloat16)
        qp_ref[:, :, 1:, 1:, :] = aq_ref[...]
        hs_ref[...] = jnp.zeros(hs_ref.shape, jnp.bfloat16)

        acc = None
        for dy in range(3):
            for dx in range(3):
                qi = 2 * ((dy + 1) % 2) + ((dx + 1) % 2)
                r0 = 0 if dy == 0 else 1
                c0 = 0 if dx == 0 else 1
                win = qp_ref[qi, :, r0:r0 + h, c0:c0 + h, :].reshape(m, c)
                part = jnp.dot(win, wds_ref[dy * 3 + dx],
                               preferred_element_type=jnp.float32)
                acc = part if acc is None else acc + part
        hcur = acc

        for b in range(nb):
            gb = gb_ref[b]
            hc16 = hcur.astype(jnp.bfloat16)
            y = jnp.dot(hc16, wcat_ref[b],
                        preferred_element_type=jnp.float32)
            y1, res = y[:, :c], y[:, c:]
            s1, t1 = _bn_scale_shift(y1, gb[0:1], gb[1:2])
            a1 = jnp.maximum(y1 * s1 + t1, 0.0).astype(jnp.bfloat16)

            hs_ref[:, 1:h + 1, 1:h + 1, :] = a1.reshape(n, h, h, c)
            acc2 = None
            for t in range(9):
                dy, dx = t // 3, t % 3
                win = hs_ref[:, dy:dy + h, dx:dx + h, :].reshape(m, c)
                part = jnp.dot(win, w2_ref[b, t],
                               preferred_element_type=jnp.float32)
                acc2 = part if acc2 is None else acc2 + part
            s2, t2 = _bn_scale_shift(acc2, gb[2:3], gb[3:4])
            a2 = jnp.maximum(acc2 * s2 + t2, 0.0).astype(jnp.bfloat16)

            y3 = jnp.dot(a2, w3_ref[b], preferred_element_type=jnp.float32)
            s3, t3 = _bn_scale_shift(y3, gb[4:5], gb[5:6])
            hcur = jnp.maximum(y3 * s3 + t3 + res, 0.0)

        o_ref[...] = hcur

    return _body


def _main_call(aq, wds, wcat, w2, w3, gb, n, h):
    c = wds.shape[2]
    nb = wcat.shape[0]
    m = n * h * h
    full = lambda shp: pl.BlockSpec(shp, lambda i: tuple(0 for _ in shp))
    return pl.pallas_call(
        _make_main_body(n, h, c, nb),
        grid=(1,),
        in_specs=[full(aq.shape), full(wds.shape), full(wcat.shape),
                  full(w2.shape), full(w3.shape), full(gb.shape)],
        out_specs=full((m, c)),
        out_shape=jax.ShapeDtypeStruct((m, c), jnp.float32),
        scratch_shapes=[
            pltpu.VMEM((4, n, h + 1, h + 1, c), jnp.bfloat16),
            pltpu.VMEM((n, h + 2, h + 2, c), jnp.bfloat16),
        ],
        compiler_params=pltpu.CompilerParams(
            dimension_semantics=("arbitrary",)),
        cost_estimate=pl.CostEstimate(
            flops=int(2 * m * c * c * (9 + 12 * nb)), transcendentals=0,
            bytes_accessed=int(2 * (m * c * 4 + (9 + 11 * nb) * c * c
                                    + 2 * m * c))),
    )(aq, wds, wcat, w2, w3, gb)


def kernel(x, stem_w, stem_g, stem_b, ds_w,
           b0_w_sc, b0_w1, b0_w2, b0_w3, b0_g1, b0_b1, b0_g2, b0_b2, b0_g3, b0_b3,
           b1_w_sc, b1_w1, b1_w2, b1_w3, b1_g1, b1_b1, b1_g2, b1_b2, b1_g3, b1_b3,
           b2_w_sc, b2_w1, b2_w2, b2_w3, b2_g1, b2_b1, b2_g2, b2_b2, b2_g3, b2_b3):
    n, cin, hi, wi = x.shape
    mid = stem_w.shape[0]
    outc = b2_w3.shape[0]
    h1 = (hi + 6 - 7) // 2 + 1
    h2 = (h1 + 2 - 3) // 2 + 1
    m1 = n * h1 * h1

    x_nhwc = jnp.transpose(x, (0, 2, 3, 1))
    xp = jnp.pad(x_nhwc, ((0, 0), (3, 3), (3, 3), (0, 0)))
    cols = [xp[:, dy:dy + 2 * h1:2, dx:dx + 2 * h1:2, :]
            for dy in range(7) for dx in range(7)]
    p1 = jnp.concatenate(cols, axis=-1).reshape(m1, 49 * cin)
    kp1 = _ceil_to(49 * cin, 128)
    p1 = jnp.pad(p1, ((0, 0), (0, kp1 - 49 * cin))).astype(jnp.bfloat16)

    ch = _ceil_to((mid + 1) // 2, 256)
    cs = 2 * ch
    w_s = jnp.transpose(stem_w, (2, 3, 1, 0)).reshape(49 * cin, mid)
    w_s = jnp.pad(w_s, ((0, kp1 - 49 * cin), (0, cs - mid))).astype(jnp.bfloat16)
    gb_s = jnp.zeros((8, cs), jnp.float32)
    gb_s = gb_s.at[0, :mid].set(stem_g).at[1, :mid].set(stem_b)

    a = _stem_call(p1, w_s, gb_s)
    a_img = a.reshape(n, h1, h1, cs)[..., :mid]

    aq = jnp.stack([a_img[:, p_::2, q_::2, :]
                    for p_ in (0, 1) for q_ in (0, 1)])

    wds = jnp.transpose(ds_w, (2, 3, 1, 0)).reshape(9, mid, mid)
    wds = wds.astype(jnp.bfloat16)

    blocks = [
        (b0_w1, b0_w_sc, b0_w2, b0_w3, b0_g1, b0_b1, b0_g2, b0_b2, b0_g3, b0_b3),
        (b1_w1, b1_w_sc, b1_w2, b1_w3, b1_g1, b1_b1, b1_g2, b1_b2, b1_g3, b1_b3),
        (b2_w1, b2_w_sc, b2_w2, b2_w3, b2_g1, b2_b1, b2_g2, b2_b2, b2_g3, b2_b3),
    ]
    wcat_l, w2_l, w3_l, gb_l = [], [], [], []
    for (w1b, wscb, w2b, w3b, g1, bb1, g2, bb2, g3, bb3) in blocks:
        wcat_l.append(jnp.concatenate(
            [w1b[:, :, 0, 0].T, wscb[:, :, 0, 0].T], axis=1))
        w2_l.append(jnp.stack([w2b[:, :, dy, dx].T
                               for dy in range(3) for dx in range(3)]))
        w3_l.append(w3b[:, :, 0, 0].T)
        zero = jnp.zeros((mid,), jnp.float32)
        gb_l.append(jnp.stack([g1, bb1, g2, bb2, g3, bb3, zero, zero]))
    wcat = jnp.stack(wcat_l).astype(jnp.bfloat16)
    w2s = jnp.stack(w2_l).astype(jnp.bfloat16)
    w3s = jnp.stack(w3_l).astype(jnp.bfloat16)
    gbs = jnp.stack(gb_l)

    out_flat = _main_call(aq, wds, wcat, w2s, w3s, gbs, n, h2)
    out = out_flat.reshape(n, h2, h2, outc)
    return jnp.transpose(out, (0, 3, 1, 2))


# speedup vs baseline: 17.8010x; 15.8620x over previous
import jax
import jax.numpy as jnp
from jax import lax
from jax.experimental import pallas as pl
from jax.experimental.pallas import tpu as pltpu

_EPS = 1e-5


def _ceil_to(v, m):
    return ((v + m - 1) // m) * m


def _bn_scale_shift(y, g, b):
    mean = jnp.mean(y, axis=0, keepdims=True)
    var = jnp.mean(y * y, axis=0, keepdims=True) - mean * mean
    scale = g * lax.rsqrt(var + _EPS)
    shift = b - mean * scale
    return scale, shift


def _stem_body(pt_ref, w_ref, gb_ref, o_ref):
    y = jnp.dot(pt_ref[...].T, w_ref[...],
                preferred_element_type=jnp.float32)
    scale, shift = _bn_scale_shift(y, gb_ref[0:1], gb_ref[1:2])
    o_ref[...] = jnp.maximum(y * scale + shift, 0.0).astype(jnp.bfloat16)


def _stem_call(pt, w_s, gb_s):
    kp1, m1 = pt.shape
    cs = w_s.shape[1]
    ch = cs // 2
    return pl.pallas_call(
        _stem_body,
        grid=(2,),
        in_specs=[
            pl.BlockSpec((kp1, m1), lambda j: (0, 0)),
            pl.BlockSpec((kp1, ch), lambda j: (0, j)),
            pl.BlockSpec((8, ch), lambda j: (0, j)),
        ],
        out_specs=pl.BlockSpec((m1, ch), lambda j: (0, j)),
        out_shape=jax.ShapeDtypeStruct((m1, cs), jnp.bfloat16),
        compiler_params=pltpu.CompilerParams(
            dimension_semantics=("parallel",)),
        cost_estimate=pl.CostEstimate(
            flops=int(2 * m1 * kp1 * cs), transcendentals=0,
            bytes_accessed=int(2 * (2 * m1 * kp1 + kp1 * cs + m1 * cs))),
    )(pt, w_s, gb_s)


def _make_main_body(n, h, c, nb):
    m = n * h * h

    def _body(aq_ref, wds_ref, wcat_ref, w2_ref, w3_ref, gb_ref, o_ref,
              qp_ref, hs_ref):
        qp_ref[...] = jnp.zeros(qp_ref.shape, jnp.bfloat16)
        qp_ref[:, :, 1:, 1:, :] = aq_ref[...]
        hs_ref[...] = jnp.zeros(hs_ref.shape, jnp.bfloat16)

        acc = None
        for dy in range(3):
            for dx in range(3):
                qi = 2 * ((dy + 1) % 2) + ((dx + 1) % 2)
                r0 = 0 if dy == 0 else 1
                c0 = 0 if dx == 0 else 1
                win = qp_ref[qi, :, r0:r0 + h, c0:c0 + h, :].reshape(m, c)
                part = jnp.dot(win, wds_ref[dy * 3 + dx],
                               preferred_element_type=jnp.float32)
                acc = part if acc is None else acc + part
        hcur = acc

        for b in range(nb):
            gb = gb_ref[b]
            hc16 = hcur.astype(jnp.bfloat16)
            y = jnp.dot(hc16, wcat_ref[b],
                        preferred_element_type=jnp.float32)
            y1, res = y[:, :c], y[:, c:]
            s1, t1 = _bn_scale_shift(y1, gb[0:1], gb[1:2])
            a1 = jnp.maximum(y1 * s1 + t1, 0.0).astype(jnp.bfloat16)

            hs_ref[:, 1:h + 1, 1:h + 1, :] = a1.reshape(n, h, h, c)
            acc2 = None
            for t in range(9):
                dy, dx = t // 3, t % 3
                win = hs_ref[:, dy:dy + h, dx:dx + h, :].reshape(m, c)
                part = jnp.dot(win, w2_ref[b, t],
                               preferred_element_type=jnp.float32)
                acc2 = part if acc2 is None else acc2 + part
            s2, t2 = _bn_scale_shift(acc2, gb[2:3], gb[3:4])
            a2 = jnp.maximum(acc2 * s2 + t2, 0.0).astype(jnp.bfloat16)

            y3 = jnp.dot(a2, w3_ref[b], preferred_element_type=jnp.float32)
            s3, t3 = _bn_scale_shift(y3, gb[4:5], gb[5:6])
            hcur = jnp.maximum(y3 * s3 + t3 + res, 0.0)

        o_ref[...] = hcur

    return _body


def _main_call(aq, wds, wcat, w2, w3, gb, n, h):
    c = wds.shape[2]
    nb = wcat.shape[0]
    m = n * h * h
    full = lambda shp: pl.BlockSpec(shp, lambda i: tuple(0 for _ in shp))
    return pl.pallas_call(
        _make_main_body(n, h, c, nb),
        grid=(1,),
        in_specs=[full(aq.shape), full(wds.shape), full(wcat.shape),
                  full(w2.shape), full(w3.shape), full(gb.shape)],
        out_specs=full((m, c)),
        out_shape=jax.ShapeDtypeStruct((m, c), jnp.float32),
        scratch_shapes=[
            pltpu.VMEM((4, n, h + 1, h + 1, c), jnp.bfloat16),
            pltpu.VMEM((n, h + 2, h + 2, c), jnp.bfloat16),
        ],
        compiler_params=pltpu.CompilerParams(
            dimension_semantics=("arbitrary",)),
        cost_estimate=pl.CostEstimate(
            flops=int(2 * m * c * c * (9 + 12 * nb)), transcendentals=0,
            bytes_accessed=int(2 * (m * c * 4 + (9 + 11 * nb) * c * c
                                    + 2 * m * c))),
    )(aq, wds, wcat, w2, w3, gb)


def kernel(x, stem_w, stem_g, stem_b, ds_w,
           b0_w_sc, b0_w1, b0_w2, b0_w3, b0_g1, b0_b1, b0_g2, b0_b2, b0_g3, b0_b3,
           b1_w_sc, b1_w1, b1_w2, b1_w3, b1_g1, b1_b1, b1_g2, b1_b2, b1_g3, b1_b3,
           b2_w_sc, b2_w1, b2_w2, b2_w3, b2_g1, b2_b1, b2_g2, b2_b2, b2_g3, b2_b3):
    n, cin, hi, wi = x.shape
    mid = stem_w.shape[0]
    outc = b2_w3.shape[0]
    h1 = (hi + 6 - 7) // 2 + 1
    h2 = (h1 + 2 - 3) // 2 + 1
    m1 = n * h1 * h1

    x_t = jnp.transpose(x, (1, 0, 2, 3))
    xp = jnp.pad(x_t, ((0, 0), (0, 0), (3, 3), (3, 3)))
    hp = hi + 6
    xr = xp.reshape(cin, n, hp // 2, 2, hp // 2, 2)
    slabs = [xr[:, :, dy // 2:dy // 2 + h1, dy % 2,
                dx // 2:dx // 2 + h1, dx % 2]
             for dy in range(7) for dx in range(7)]
    pt = jnp.stack(slabs, axis=0).reshape(49 * cin, m1)
    kp1 = _ceil_to(49 * cin, 8)
    pt = jnp.pad(pt, ((0, kp1 - 49 * cin), (0, 0))).astype(jnp.bfloat16)

    ch = _ceil_to((mid + 1) // 2, 256)
    cs = 2 * ch
    w_s = jnp.transpose(stem_w, (2, 3, 1, 0)).reshape(49 * cin, mid)
    w_s = jnp.pad(w_s, ((0, kp1 - 49 * cin), (0, cs - mid))).astype(jnp.bfloat16)
    gb_s = jnp.zeros((8, cs), jnp.float32)
    gb_s = gb_s.at[0, :mid].set(stem_g).at[1, :mid].set(stem_b)

    a = _stem_call(pt, w_s, gb_s)

    ar = a.reshape(n, h1 // 2, 2, h1 // 2, 2, cs)[..., :mid]
    aq = jnp.stack([ar[:, :, p_, :, q_, :]
                    for p_ in (0, 1) for q_ in (0, 1)])

    wds = jnp.transpose(ds_w, (2, 3, 1, 0)).reshape(9, mid, mid)
    wds = wds.astype(jnp.bfloat16)

    blocks = [
        (b0_w1, b0_w_sc, b0_w2, b0_w3, b0_g1, b0_b1, b0_g2, b0_b2, b0_g3, b0_b3),
        (b1_w1, b1_w_sc, b1_w2, b1_w3, b1_g1, b1_b1, b1_g2, b1_b2, b1_g3, b1_b3),
        (b2_w1, b2_w_sc, b2_w2, b2_w3, b2_g1, b2_b1, b2_g2, b2_b2, b2_g3, b2_b3),
    ]
    wcat_l, w2_l, w3_l, gb_l = [], [], [], []
    for (w1b, wscb, w2b, w3b, g1, bb1, g2, bb2, g3, bb3) in blocks:
        wcat_l.append(jnp.concatenate(
            [w1b[:, :, 0, 0].T, wscb[:, :, 0, 0].T], axis=1))
        w2_l.append(jnp.stack([w2b[:, :, dy, dx].T
                               for dy in range(3) for dx in range(3)]))
        w3_l.append(w3b[:, :, 0, 0].T)
        zero = jnp.zeros((mid,), jnp.float32)
        gb_l.append(jnp.stack([g1, bb1, g2, bb2, g3, bb3, zero, zero]))
    wcat = jnp.stack(wcat_l).astype(jnp.bfloat16)
    w2s = jnp.stack(w2_l).astype(jnp.bfloat16)
    w3s = jnp.stack(w3_l).astype(jnp.bfloat16)
    gbs = jnp.stack(gb_l)

    out_flat = _main_call(aq, wds, wcat, w2s, w3s, gbs, n, h2)
    out = out_flat.reshape(n, h2, h2, outc)
    return jnp.transpose(out, (0, 3, 1, 2))


# speedup vs baseline: 27.2692x; 1.5319x over previous
import jax
import jax.numpy as jnp
from jax import lax
from jax.experimental import pallas as pl
from jax.experimental.pallas import tpu as pltpu

_EPS = 1e-5


def _ceil_to(v, m):
    return ((v + m - 1) // m) * m


def _bn_scale_shift(y, g, b):
    mean = jnp.mean(y, axis=0, keepdims=True)
    var = jnp.mean(y * y, axis=0, keepdims=True) - mean * mean
    scale = g * lax.rsqrt(var + _EPS)
    shift = b - mean * scale
    return scale, shift


def _stem_body(pt_ref, w_ref, gb_ref, o_ref):
    y = jnp.dot(pt_ref[...].T, w_ref[...],
                preferred_element_type=jnp.float32)
    scale, shift = _bn_scale_shift(y, gb_ref[0:1], gb_ref[1:2])
    o_ref[...] = jnp.maximum(y * scale + shift, 0.0).astype(jnp.bfloat16)


def _stem_call(pt, w_s, gb_s):
    kp1, m1 = pt.shape
    cs = w_s.shape[1]
    ch = cs // 2
    return pl.pallas_call(
        _stem_body,
        grid=(2,),
        in_specs=[
            pl.BlockSpec((kp1, m1), lambda j: (0, 0)),
            pl.BlockSpec((kp1, ch), lambda j: (0, j)),
            pl.BlockSpec((8, ch), lambda j: (0, j)),
        ],
        out_specs=pl.BlockSpec((m1, ch), lambda j: (0, j)),
        out_shape=jax.ShapeDtypeStruct((m1, cs), jnp.bfloat16),
        compiler_params=pltpu.CompilerParams(
            dimension_semantics=("parallel",)),
        cost_estimate=pl.CostEstimate(
            flops=int(2 * m1 * kp1 * cs), transcendentals=0,
            bytes_accessed=int(2 * (2 * m1 * kp1 + kp1 * cs + m1 * cs))),
    )(pt, w_s, gb_s)


def _make_main_body(n, h, c, nb):
    m = n * h * h

    def _body(aq_ref, wds_ref, wcat_ref, w2_ref, w3_ref, gb_ref, o_ref,
              qp_ref, hs_ref):
        qp_ref[...] = jnp.zeros(qp_ref.shape, jnp.bfloat16)
        qp_ref[:, :, 1:, 1:, :] = aq_ref[...]
        hs_ref[...] = jnp.zeros(hs_ref.shape, jnp.bfloat16)

        acc = None
        for dy in range(3):
            for dx in range(3):
                qi = 2 * ((dy + 1) % 2) + ((dx + 1) % 2)
                r0 = 0 if dy == 0 else 1
                c0 = 0 if dx == 0 else 1
                win = qp_ref[qi, :, r0:r0 + h, c0:c0 + h, :].reshape(m, c)
                part = jnp.dot(win, wds_ref[dy * 3 + dx],
                               preferred_element_type=jnp.float32)
                acc = part if acc is None else acc + part
        hcur = acc

        for b in range(nb):
            gb = gb_ref[b]
            hc16 = hcur.astype(jnp.bfloat16)
            y = jnp.dot(hc16, wcat_ref[b],
                        preferred_element_type=jnp.float32)
            y1, res = y[:, :c], y[:, c:]
            s1, t1 = _bn_scale_shift(y1, gb[0:1], gb[1:2])
            a1 = jnp.maximum(y1 * s1 + t1, 0.0).astype(jnp.bfloat16)

            hs_ref[:, 1:h + 1, 1:h + 1, :] = a1.reshape(n, h, h, c)
            acc2 = None
            for t in range(9):
                dy, dx = t // 3, t % 3
                win = hs_ref[:, dy:dy + h, dx:dx + h, :].reshape(m, c)
                part = jnp.dot(win, w2_ref[b, t],
                               preferred_element_type=jnp.float32)
                acc2 = part if acc2 is None else acc2 + part
            s2, t2 = _bn_scale_shift(acc2, gb[2:3], gb[3:4])
            a2 = jnp.maximum(acc2 * s2 + t2, 0.0).astype(jnp.bfloat16)

            y3 = jnp.dot(a2, w3_ref[b], preferred_element_type=jnp.float32)
            s3, t3 = _bn_scale_shift(y3, gb[4:5], gb[5:6])
            hcur = jnp.maximum(y3 * s3 + t3 + res, 0.0)

        o_ref[...] = hcur

    return _body


def _main_call(aq, wds, wcat, w2, w3, gb, n, h):
    c = wds.shape[2]
    nb = wcat.shape[0]
    m = n * h * h
    full = lambda shp: pl.BlockSpec(shp, lambda i: tuple(0 for _ in shp))
    return pl.pallas_call(
        _make_main_body(n, h, c, nb),
        grid=(1,),
        in_specs=[full(aq.shape), full(wds.shape), full(wcat.shape),
                  full(w2.shape), full(w3.shape), full(gb.shape)],
        out_specs=full((m, c)),
        out_shape=jax.ShapeDtypeStruct((m, c), jnp.float32),
        scratch_shapes=[
            pltpu.VMEM((4, n, h + 1, h + 1, c), jnp.bfloat16),
            pltpu.VMEM((n, h + 2, h + 2, c), jnp.bfloat16),
        ],
        compiler_params=pltpu.CompilerParams(
            dimension_semantics=("arbitrary",)),
        cost_estimate=pl.CostEstimate(
            flops=int(2 * m * c * c * (9 + 12 * nb)), transcendentals=0,
            bytes_accessed=int(2 * (m * c * 4 + (9 + 11 * nb) * c * c
                                    + 2 * m * c))),
    )(aq, wds, wcat, w2, w3, gb)


def kernel(x, stem_w, stem_g, stem_b, ds_w,
           b0_w_sc, b0_w1, b0_w2, b0_w3, b0_g1, b0_b1, b0_g2, b0_b2, b0_g3, b0_b3,
           b1_w_sc, b1_w1, b1_w2, b1_w3, b1_g1, b1_b1, b1_g2, b1_b2, b1_g3, b1_b3,
           b2_w_sc, b2_w1, b2_w2, b2_w3, b2_g1, b2_b1, b2_g2, b2_b2, b2_g3, b2_b3):
    n, cin, hi, wi = x.shape
    mid = stem_w.shape[0]
    outc = b2_w3.shape[0]
    h1 = (hi + 6 - 7) // 2 + 1
    h2 = (h1 + 2 - 3) // 2 + 1
    m1 = n * h1 * h1

    x_t = jnp.transpose(x, (1, 0, 2, 3))
    xo = x_t[..., 1::2]
    xe = x_t[..., 0::2]
    wq = hi // 2 + 3
    wp0 = jnp.pad(xo, ((0, 0), (0, 0), (3, 3), (2, wq - hi // 2 - 2)))
    wp1 = jnp.pad(xe, ((0, 0), (0, 0), (3, 3), (1, wq - hi // 2 - 1)))
    wps = (wp0, wp1)
    slabs = [wps[dx % 2][:, :, dy:dy + 2 * h1:2,
                         dx // 2:dx // 2 + h1]
             for dy in range(7) for dx in range(7)]
    pt = jnp.stack(slabs, axis=0).reshape(49 * cin, m1)
    kp1 = _ceil_to(49 * cin, 8)
    pt = jnp.pad(pt, ((0, kp1 - 49 * cin), (0, 0))).astype(jnp.bfloat16)

    ch = _ceil_to((mid + 1) // 2, 256)
    cs = 2 * ch
    w_s = jnp.transpose(stem_w, (2, 3, 1, 0)).reshape(49 * cin, mid)
    w_s = jnp.pad(w_s, ((0, kp1 - 49 * cin), (0, cs - mid))).astype(jnp.bfloat16)
    gb_s = jnp.zeros((8, cs), jnp.float32)
    gb_s = gb_s.at[0, :mid].set(stem_g).at[1, :mid].set(stem_b)

    a = _stem_call(pt, w_s, gb_s)

    ar = a.reshape(n, h1 // 2, 2, h1 // 2, 2, cs)[..., :mid]
    aq = jnp.stack([ar[:, :, p_, :, q_, :]
                    for p_ in (0, 1) for q_ in (0, 1)])

    wds = jnp.transpose(ds_w, (2, 3, 1, 0)).reshape(9, mid, mid)
    wds = wds.astype(jnp.bfloat16)

    blocks = [
        (b0_w1, b0_w_sc, b0_w2, b0_w3, b0_g1, b0_b1, b0_g2, b0_b2, b0_g3, b0_b3),
        (b1_w1, b1_w_sc, b1_w2, b1_w3, b1_g1, b1_b1, b1_g2, b1_b2, b1_g3, b1_b3),
        (b2_w1, b2_w_sc, b2_w2, b2_w3, b2_g1, b2_b1, b2_g2, b2_b2, b2_g3, b2_b3),
    ]
    wcat_l, w2_l, w3_l, gb_l = [], [], [], []
    for (w1b, wscb, w2b, w3b, g1, bb1, g2, bb2, g3, bb3) in blocks:
        wcat_l.append(jnp.concatenate(
            [w1b[:, :, 0, 0].T, wscb[:, :, 0, 0].T], axis=1))
        w2_l.append(jnp.stack([w2b[:, :, dy, dx].T
                               for dy in range(3) for dx in range(3)]))
        w3_l.append(w3b[:, :, 0, 0].T)
        zero = jnp.zeros((mid,), jnp.float32)
        gb_l.append(jnp.stack([g1, bb1, g2, bb2, g3, bb3, zero, zero]))
    wcat = jnp.stack(wcat_l).astype(jnp.bfloat16)
    w2s = jnp.stack(w2_l).astype(jnp.bfloat16)
    w3s = jnp.stack(w3_l).astype(jnp.bfloat16)
    gbs = jnp.stack(gb_l)

    out_flat = _main_call(aq, wds, wcat, w2s, w3s, gbs, n, h2)
    out = out_flat.reshape(n, h2, h2, outc)
    return jnp.transpose(out, (0, 3, 1, 2))


# speedup vs baseline: 29.1219x; 1.0679x over previous
import jax
import jax.numpy as jnp
from jax import lax
from jax.experimental import pallas as pl
from jax.experimental.pallas import tpu as pltpu

_EPS = 1e-5


def _ceil_to(v, m):
    return ((v + m - 1) // m) * m


def _bn_scale_shift(y, g, b):
    mean = jnp.mean(y, axis=0, keepdims=True)
    var = jnp.mean(y * y, axis=0, keepdims=True) - mean * mean
    scale = g * lax.rsqrt(var + _EPS)
    shift = b - mean * scale
    return scale, shift


def _stem_body(pt_ref, w_ref, gb_ref, o_ref):
    y = jnp.dot(pt_ref[...].T, w_ref[...],
                preferred_element_type=jnp.float32)
    scale, shift = _bn_scale_shift(y, gb_ref[0:1], gb_ref[1:2])
    o_ref[...] = jnp.maximum(y * scale + shift, 0.0).astype(jnp.bfloat16)


def _stem_call(pt, w_s, gb_s):
    kp1, m1 = pt.shape
    cs = w_s.shape[1]
    ch = cs // 2
    return pl.pallas_call(
        _stem_body,
        grid=(2,),
        in_specs=[
            pl.BlockSpec((kp1, m1), lambda j: (0, 0)),
            pl.BlockSpec((kp1, ch), lambda j: (0, j)),
            pl.BlockSpec((8, ch), lambda j: (0, j)),
        ],
        out_specs=pl.BlockSpec((m1, ch), lambda j: (0, j)),
        out_shape=jax.ShapeDtypeStruct((m1, cs), jnp.bfloat16),
        compiler_params=pltpu.CompilerParams(
            dimension_semantics=("parallel",)),
        cost_estimate=pl.CostEstimate(
            flops=int(2 * m1 * kp1 * cs), transcendentals=0,
            bytes_accessed=int(2 * (2 * m1 * kp1 + kp1 * cs + m1 * cs))),
    )(pt, w_s, gb_s)


def _make_main_body(n, h, c, nb):
    m = n * h * h

    def _body(aq_ref, wds_ref, wcat_ref, w2_ref, w3_ref, gb_ref, o_ref,
              qp_ref, hs_ref):
        qp_ref[...] = jnp.zeros(qp_ref.shape, jnp.bfloat16)
        qp_ref[:, :, 1:, 1:, :] = aq_ref[...]
        hs_ref[...] = jnp.zeros(hs_ref.shape, jnp.bfloat16)

        acc = None
        for dy in range(3):
            for dx in range(3):
                qi = 2 * ((dy + 1) % 2) + ((dx + 1) % 2)
                r0 = 0 if dy == 0 else 1
                c0 = 0 if dx == 0 else 1
                win = qp_ref[qi, :, r0:r0 + h, c0:c0 + h, :].reshape(m, c)
                part = jnp.dot(win, wds_ref[dy * 3 + dx],
                               preferred_element_type=jnp.float32)
                acc = part if acc is None else acc + part
        hcur = acc

        for b in range(nb):
            gb = gb_ref[b]
            hc16 = hcur.astype(jnp.bfloat16)
            y = jnp.dot(hc16, wcat_ref[b],
                        preferred_element_type=jnp.float32)
            y1, res = y[:, :c], y[:, c:]
            s1, t1 = _bn_scale_shift(y1, gb[0:1], gb[1:2])
            a1 = jnp.maximum(y1 * s1 + t1, 0.0).astype(jnp.bfloat16)

            hs_ref[:, 1:h + 1, 1:h + 1, :] = a1.reshape(n, h, h, c)
            acc2 = None
            for t in range(9):
                dy, dx = t // 3, t % 3
                win = hs_ref[:, dy:dy + h, dx:dx + h, :].reshape(m, c)
                part = jnp.dot(win, w2_ref[b, t],
                               preferred_element_type=jnp.float32)
                acc2 = part if acc2 is None else acc2 + part
            s2, t2 = _bn_scale_shift(acc2, gb[2:3], gb[3:4])
            a2 = jnp.maximum(acc2 * s2 + t2, 0.0).astype(jnp.bfloat16)

            y3 = jnp.dot(a2, w3_ref[b], preferred_element_type=jnp.float32)
            s3, t3 = _bn_scale_shift(y3, gb[4:5], gb[5:6])
            hcur = jnp.maximum(y3 * s3 + t3 + res, 0.0)

        o_ref[...] = hcur

    return _body


def _main_call(aq, wds, wcat, w2, w3, gb, n, h):
    c = wds.shape[2]
    nb = wcat.shape[0]
    m = n * h * h
    full = lambda shp: pl.BlockSpec(shp, lambda i: tuple(0 for _ in shp))
    return pl.pallas_call(
        _make_main_body(n, h, c, nb),
        grid=(1,),
        in_specs=[full(aq.shape), full(wds.shape), full(wcat.shape),
                  full(w2.shape), full(w3.shape), full(gb.shape)],
        out_specs=full((m, c)),
        out_shape=jax.ShapeDtypeStruct((m, c), jnp.float32),
        scratch_shapes=[
            pltpu.VMEM((4, n, h + 1, h + 1, c), jnp.bfloat16),
            pltpu.VMEM((n, h + 2, h + 2, c), jnp.bfloat16),
        ],
        compiler_params=pltpu.CompilerParams(
            dimension_semantics=("arbitrary",)),
        cost_estimate=pl.CostEstimate(
            flops=int(2 * m * c * c * (9 + 12 * nb)), transcendentals=0,
            bytes_accessed=int(2 * (m * c * 4 + (9 + 11 * nb) * c * c
                                    + 2 * m * c))),
    )(aq, wds, wcat, w2, w3, gb)


def kernel(x, stem_w, stem_g, stem_b, ds_w,
           b0_w_sc, b0_w1, b0_w2, b0_w3, b0_g1, b0_b1, b0_g2, b0_b2, b0_g3, b0_b3,
           b1_w_sc, b1_w1, b1_w2, b1_w3, b1_g1, b1_b1, b1_g2, b1_b2, b1_g3, b1_b3,
           b2_w_sc, b2_w1, b2_w2, b2_w3, b2_g1, b2_b1, b2_g2, b2_b2, b2_g3, b2_b3):
    n, cin, hi, wi = x.shape
    mid = stem_w.shape[0]
    outc = b2_w3.shape[0]
    h1 = (hi + 6 - 7) // 2 + 1
    h2 = (h1 + 2 - 3) // 2 + 1
    m1 = n * h1 * h1

    x_t = jnp.transpose(x, (1, 0, 2, 3))
    xo = x_t[..., 1::2]
    xe = x_t[..., 0::2]
    wq = hi // 2 + 3
    wp0 = jnp.pad(xo, ((0, 0), (0, 0), (3, 3), (2, wq - hi // 2 - 2)))
    wp1 = jnp.pad(xe, ((0, 0), (0, 0), (3, 3), (1, wq - hi // 2 - 1)))
    p4 = [[wp[:, :, p_::2, :] for wp in (wp0, wp1)] for p_ in (0, 1)]
    slabs = [p4[dy % 2][dx % 2][:, :, dy // 2:dy // 2 + h1,
                                dx // 2:dx // 2 + h1]
             for dy in range(7) for dx in range(7)]
    pt = jnp.stack(slabs, axis=0).reshape(49 * cin, m1)
    kp1 = _ceil_to(49 * cin, 8)
    pt = jnp.pad(pt, ((0, kp1 - 49 * cin), (0, 0))).astype(jnp.bfloat16)

    ch = _ceil_to((mid + 1) // 2, 256)
    cs = 2 * ch
    w_s = jnp.transpose(stem_w, (2, 3, 1, 0)).reshape(49 * cin, mid)
    w_s = jnp.pad(w_s, ((0, kp1 - 49 * cin), (0, cs - mid))).astype(jnp.bfloat16)
    gb_s = jnp.zeros((8, cs), jnp.float32)
    gb_s = gb_s.at[0, :mid].set(stem_g).at[1, :mid].set(stem_b)

    a = _stem_call(pt, w_s, gb_s)

    ar = a.reshape(n, h1 // 2, 2, h1 // 2, 2, cs)[..., :mid]
    aq = jnp.stack([ar[:, :, p_, :, q_, :]
                    for p_ in (0, 1) for q_ in (0, 1)])

    wds = jnp.transpose(ds_w, (2, 3, 1, 0)).reshape(9, mid, mid)
    wds = wds.astype(jnp.bfloat16)

    blocks = [
        (b0_w1, b0_w_sc, b0_w2, b0_w3, b0_g1, b0_b1, b0_g2, b0_b2, b0_g3, b0_b3),
        (b1_w1, b1_w_sc, b1_w2, b1_w3, b1_g1, b1_b1, b1_g2, b1_b2, b1_g3, b1_b3),
        (b2_w1, b2_w_sc, b2_w2, b2_w3, b2_g1, b2_b1, b2_g2, b2_b2, b2_g3, b2_b3),
    ]
    wcat_l, w2_l, w3_l, gb_l = [], [], [], []
    for (w1b, wscb, w2b, w3b, g1, bb1, g2, bb2, g3, bb3) in blocks:
        wcat_l.append(jnp.concatenate(
            [w1b[:, :, 0, 0].T, wscb[:, :, 0, 0].T], axis=1))
        w2_l.append(jnp.stack([w2b[:, :, dy, dx].T
                               for dy in range(3) for dx in range(3)]))
        w3_l.append(w3b[:, :, 0, 0].T)
        zero = jnp.zeros((mid,), jnp.float32)
        gb_l.append(jnp.stack([g1, bb1, g2, bb2, g3, bb3, zero, zero]))
    wcat = jnp.stack(wcat_l).astype(jnp.bfloat16)
    w2s = jnp.stack(w2_l).astype(jnp.bfloat16)
    w3s = jnp.stack(w3_l).astype(jnp.bfloat16)
    gbs = jnp.stack(gb_l)

    out_flat = _main_call(aq, wds, wcat, w2s, w3s, gbs, n, h2)
    out = out_flat.reshape(n, h2, h2, outc)
    return jnp.transpose(out, (0, 3, 1, 2))
